# Initial kernel scaffold; baseline (speedup 1.0000x reference)
import jax
import jax.numpy as jnp
from jax import lax
from jax.experimental import pallas as pl
from jax.experimental.pallas import tpu as pltpu
from jax.experimental.pallas import tpu_sc as plsc

N = 10000
E = 160000
DIN = 256
DOUT = 256
H = 128
NT = 16
L = 16
EPT = E // NT
G = 128
CH = -(-EPT // G)
EPTP = CH * G
NP = 10240
RPT = NP // NT
BN = 1000


def _tc_body(x_ref, wt_ref, a1_ref, a2_ref, z0_ref, z1_ref, s_ref, d_ref):
    z = jnp.dot(x_ref[...], wt_ref[...], preferred_element_type=jnp.float32)
    z0_ref[...] = z[:, :H]
    z1_ref[...] = z[:, H:]
    s_ref[...] = jnp.dot(z, a1_ref[...], preferred_element_type=jnp.float32)
    d_ref[...] = jnp.dot(z, a2_ref[...], preferred_element_type=jnp.float32)


def _sc_body(z0_h, z1_h, s_h, d_h, src_h, dst_h, out0_h, out1_h,
             s_v, d_v, src_v, dst_v, ee_v, den_v, inv_v, part_v, invsl_v,
             alpha_v, gbuf, hacc_s, denslab_s, inv_s, sem):
    cid = lax.axis_index("c")
    sid = lax.axis_index("s")

    pltpu.sync_copy(s_h, s_v)
    pltpu.sync_copy(d_h, d_v)
    pltpu.sync_copy(src_h.at[sid], src_v)
    pltpu.sync_copy(dst_h.at[sid], dst_v)

    zero16 = jnp.zeros((L,), jnp.float32)

    def zero_den(i, c):
        den_v[pl.ds(i * L, L)] = zero16
        return c
    lax.fori_loop(0, NP // L, zero_den, 0)

    def zero_g(r, c):
        for v in range(H // L):
            gbuf[r, pl.ds(v * L, L)] = zero16
        return c
    lax.fori_loop(0, G, zero_g, 0)
    for b in range(RPT // G):
        pltpu.sync_copy(gbuf, hacc_s.at[pl.ds(sid * RPT + b * G, G)])

    neg = jnp.full((L,), -1e30, jnp.float32)

    def mxs(i, acc):
        return jnp.maximum(acc, s_v[pl.ds(i * L, L)])

    def mxd(i, acc):
        return jnp.maximum(acc, d_v[pl.ds(i * L, L)])
    M = jnp.max(lax.fori_loop(0, NP // L, mxs, neg)) + \
        jnp.max(lax.fori_loop(0, NP // L, mxd, neg))

    def ph1(j, c):
        for k in range(G // L):
            sl = pl.ds(k * L, L)
            s16 = src_v[j, sl]
            d16 = dst_v[j, sl]
            t = plsc.load_gather(s_v, [s16]) + plsc.load_gather(d_v, [d16])
            e = jnp.where(t >= 0, t, 0.01 * t)
            ee = jnp.exp(e - M)
            lidx = j * G + k * L + lax.iota(jnp.int32, L)
            ee = jnp.where(lidx < EPT, ee, 0.0)
            ee_v[j, sl] = ee
            plsc.addupdate_scatter(den_v, [d16], ee)
        return c
    lax.fori_loop(0, CH, ph1, 0)

    pltpu.sync_copy(den_v, denslab_s.at[sid])
    plsc.subcore_barrier()
    for i in range(NT):
        pltpu.sync_copy(denslab_s.at[i, pl.ds(sid * RPT, RPT)], part_v.at[i])

    def red(v, c):
        sl = pl.ds(v * L, L)
        acc = part_v[0, sl]
        for i in range(1, NT):
            acc = acc + part_v[i, sl]
        invsl_v[sl] = jnp.where(acc > 0, 1.0 / acc, 1.0)
        return c
    lax.fori_loop(0, RPT // L, red, 0)
    pltpu.sync_copy(invsl_v, inv_s.at[pl.ds(sid * RPT, RPT)])
    plsc.subcore_barrier()
    pltpu.sync_copy(inv_s, inv_v)

    def phase2(z_h):
        def ph2(j, c):
            cp = pltpu.async_copy(z_h.at[src_v.at[j]], gbuf, sem)
            for k in range(G // L):
                sl = pl.ds(k * L, L)
                iv = plsc.load_gather(inv_v, [dst_v[j, sl]])
                alpha_v[sl] = ee_v[j, sl] * iv
            cp.wait()

            def scale(r, cc):
                ab = jnp.full((L,), alpha_v[r])
                for v in range(H // L):
                    sl2 = pl.ds(v * L, L)
                    gbuf[r, sl2] = gbuf[r, sl2] * ab
                return cc
            lax.fori_loop(0, G, scale, 0)
            pltpu.async_copy(gbuf, hacc_s.at[dst_v.at[j]], sem, add=True).wait()
            return c
        lax.fori_loop(0, CH, ph2, 0)

    @pl.when(cid == 0)
    def _():
        phase2(z0_h)

    @pl.when(cid == 1)
    def _():
        phase2(z1_h)

    plsc.subcore_barrier()

    @pl.when(cid == 0)
    def _():
        pltpu.sync_copy(hacc_s.at[pl.ds(sid * RPT, RPT)],
                        out0_h.at[pl.ds(sid * RPT, RPT)])

    @pl.when(cid == 1)
    def _():
        pltpu.sync_copy(hacc_s.at[pl.ds(sid * RPT, RPT)],
                        out1_h.at[pl.ds(sid * RPT, RPT)])


def kernel(x, edge_index, W, A):
    Wt = W.T
    a1 = A[0, :DOUT].reshape(DOUT, 1)
    a2 = A[0, DOUT:].reshape(DOUT, 1)
    z0, z1, s2, d2 = pl.pallas_call(
        _tc_body,
        grid=(N // BN,),
        in_specs=[pl.BlockSpec((BN, DIN), lambda i: (i, 0)),
                  pl.BlockSpec((DIN, DOUT), lambda i: (0, 0)),
                  pl.BlockSpec((DOUT, 1), lambda i: (0, 0)),
                  pl.BlockSpec((DOUT, 1), lambda i: (0, 0))],
        out_specs=[pl.BlockSpec((BN, H), lambda i: (i, 0)),
                   pl.BlockSpec((BN, H), lambda i: (i, 0)),
                   pl.BlockSpec((BN, 1), lambda i: (i, 0)),
                   pl.BlockSpec((BN, 1), lambda i: (i, 0))],
        out_shape=[jax.ShapeDtypeStruct((N, H), jnp.float32),
                   jax.ShapeDtypeStruct((N, H), jnp.float32),
                   jax.ShapeDtypeStruct((N, 1), jnp.float32),
                   jax.ShapeDtypeStruct((N, 1), jnp.float32)],
    )(x, Wt, a1, a2)

    s = jnp.pad(s2[:, 0], (0, NP - N))
    d = jnp.pad(d2[:, 0], (0, NP - N))
    src = jnp.pad(edge_index[0].reshape(NT, EPT),
                  ((0, 0), (0, EPTP - EPT))).reshape(NT, CH, G)
    dst = jnp.pad(edge_index[1].reshape(NT, EPT),
                  ((0, 0), (0, EPTP - EPT))).reshape(NT, CH, G)

    sc = pl.kernel(
        _sc_body,
        out_type=[jax.ShapeDtypeStruct((NP, H), jnp.float32),
                  jax.ShapeDtypeStruct((NP, H), jnp.float32)],
        mesh=plsc.VectorSubcoreMesh(core_axis_name="c", subcore_axis_name="s"),
        scratch_types=[
            pltpu.VMEM((NP,), jnp.float32),
            pltpu.VMEM((NP,), jnp.float32),
            pltpu.VMEM((CH, G), jnp.int32),
            pltpu.VMEM((CH, G), jnp.int32),
            pltpu.VMEM((CH, G), jnp.float32),
            pltpu.VMEM((NP,), jnp.float32),
            pltpu.VMEM((NP,), jnp.float32),
            pltpu.VMEM((NT, RPT), jnp.float32),
            pltpu.VMEM((RPT,), jnp.float32),
            pltpu.VMEM((G,), jnp.float32),
            pltpu.VMEM((G, H), jnp.float32),
            pltpu.VMEM_SHARED((NP, H), jnp.float32),
            pltpu.VMEM_SHARED((NT, NP), jnp.float32),
            pltpu.VMEM_SHARED((NP,), jnp.float32),
            pltpu.SemaphoreType.DMA,
        ],
    )
    out0, out1 = sc(z0, z1, s, d, src, dst)
    return jnp.concatenate([out0[:N], out1[:N]], axis=1)


# speedup vs baseline: 5.0017x; 5.0017x over previous
import jax
import jax.numpy as jnp
from jax import lax
from jax.experimental import pallas as pl
from jax.experimental.pallas import tpu as pltpu
from jax.experimental.pallas import tpu_sc as plsc

N = 10000
E = 160000
DIN = 256
DOUT = 256
Q = 64
NT = 16
L = 16
EPT = E // NT
G = 128
CH = -(-EPT // G)
EPTP = CH * G
NP = 10240
RPT = NP // NT
NB = RPT // G
BN = 1000


def _tc_body(x_ref, wt_ref, a1_ref, a2_ref,
             z0_ref, z1_ref, z2_ref, z3_ref, s_ref, d_ref):
    z = jnp.dot(x_ref[...], wt_ref[...], preferred_element_type=jnp.float32)
    z0_ref[...] = z[:, 0 * Q:1 * Q]
    z1_ref[...] = z[:, 1 * Q:2 * Q]
    z2_ref[...] = z[:, 2 * Q:3 * Q]
    z3_ref[...] = z[:, 3 * Q:4 * Q]
    s_ref[...] = jnp.dot(z, a1_ref[...], preferred_element_type=jnp.float32)
    d_ref[...] = jnp.dot(z, a2_ref[...], preferred_element_type=jnp.float32)


def _sc_body(z0_h, z1_h, z2_h, z3_h, s_h, d_h, src_h, dst_h,
             o0_h, o1_h, o2_h, o3_h,
             s_v, d_v, src_v, dst_v, ee_v, invsl_v, alpha_v,
             gbuf, hacc_s, den_s, sem):
    cid = lax.axis_index("c")
    sid = lax.axis_index("s")

    pltpu.sync_copy(s_h, s_v)
    pltpu.sync_copy(d_h, d_v)
    pltpu.sync_copy(src_h.at[sid], src_v)
    pltpu.sync_copy(dst_h.at[sid], dst_v)

    zero16 = jnp.zeros((L,), jnp.float32)
    iota16 = lax.iota(jnp.int32, L)

    def zero_invsl(i, c):
        invsl_v[pl.ds(i * L, L)] = zero16
        return c
    lax.fori_loop(0, RPT // L, zero_invsl, 0)
    pltpu.sync_copy(invsl_v, den_s.at[pl.ds(sid * RPT, RPT)])

    neg = jnp.full((L,), -1e30, jnp.float32)

    def mxs(i, acc):
        return jnp.maximum(acc, s_v[pl.ds(i * L, L)])

    def mxd(i, acc):
        return jnp.maximum(acc, d_v[pl.ds(i * L, L)])

    def lane_max(v):
        m = v[0]
        for i in range(1, L):
            m = jnp.maximum(m, v[i])
        return m
    M = lane_max(lax.fori_loop(0, NP // L, mxs, neg)) + \
        lane_max(lax.fori_loop(0, NP // L, mxd, neg))

    plsc.subcore_barrier()

    def ph1(j, c):
        for k in range(G // L):
            sl = pl.ds(k * L, L)
            s16 = src_v[j, sl]
            d16 = dst_v[j, sl]
            t = plsc.load_gather(s_v, [s16]) + plsc.load_gather(d_v, [d16])
            e = jnp.where(t >= 0, t, 0.01 * t)
            ee = jnp.exp(e - M)
            lidx = j * G + k * L + iota16
            ee = jnp.where(lidx < EPT, ee, 0.0)
            ee_v[j, sl] = ee
        pltpu.async_copy(ee_v.at[j], den_s.at[dst_v.at[j]], sem,
                         add=True).wait()
        return c
    lax.fori_loop(0, CH, ph1, 0)
    plsc.subcore_barrier()
    pltpu.sync_copy(den_s.at[pl.ds(sid * RPT, RPT)], invsl_v)

    def inv_loop(v, c):
        sl = pl.ds(v * L, L)
        acc = invsl_v[sl]
        invsl_v[sl] = jnp.where(acc > 0, 1.0 / acc, 1.0)
        return c
    lax.fori_loop(0, RPT // L, inv_loop, 0)
    pltpu.sync_copy(invsl_v, den_s.at[pl.ds(sid * RPT, RPT)])
    plsc.subcore_barrier()
    pltpu.sync_copy(den_s, s_v)

    def zero_gbuf():
        def zg(r, c):
            for v in range(Q // L):
                gbuf[r, pl.ds(v * L, L)] = zero16
            return c
        lax.fori_loop(0, G, zg, 0)

    def phase2(z_h, out_h):
        zero_gbuf()
        for b in range(NB):
            pltpu.sync_copy(gbuf, hacc_s.at[pl.ds(sid * RPT + b * G, G)])
        plsc.subcore_barrier()

        def ph2(j, c):
            cp = pltpu.async_copy(z_h.at[src_v.at[j]], gbuf, sem)
            for k in range(G // L):
                sl = pl.ds(k * L, L)
                iv = plsc.load_gather(s_v, [dst_v[j, sl]])
                alpha_v[sl] = ee_v[j, sl] * iv
            cp.wait()

            def scale(g, cc):
                a16 = alpha_v[pl.ds(g * L, L)]
                for r in range(L):
                    ab = jnp.full((L,), a16[r])
                    row = g * L + r
                    for v in range(Q // L):
                        sl2 = pl.ds(v * L, L)
                        gbuf[row, sl2] = gbuf[row, sl2] * ab
                return cc
            lax.fori_loop(0, G // L, scale, 0)
            pltpu.async_copy(gbuf, hacc_s.at[dst_v.at[j]], sem, add=True).wait()
            return c
        lax.fori_loop(0, CH, ph2, 0)
        plsc.subcore_barrier()
        pltpu.sync_copy(hacc_s.at[pl.ds(sid * RPT, RPT)],
                        out_h.at[pl.ds(sid * RPT, RPT)])

    @pl.when(cid == 0)
    def _():
        phase2(z0_h, o0_h)
        phase2(z1_h, o1_h)

    @pl.when(cid == 1)
    def _():
        phase2(z2_h, o2_h)
        phase2(z3_h, o3_h)


def kernel(x, edge_index, W, A):
    Wt = W.T
    a1 = A[0, :DOUT].reshape(DOUT, 1)
    a2 = A[0, DOUT:].reshape(DOUT, 1)
    zq = pl.pallas_call(
        _tc_body,
        grid=(N // BN,),
        in_specs=[pl.BlockSpec((BN, DIN), lambda i: (i, 0)),
                  pl.BlockSpec((DIN, DOUT), lambda i: (0, 0)),
                  pl.BlockSpec((DOUT, 1), lambda i: (0, 0)),
                  pl.BlockSpec((DOUT, 1), lambda i: (0, 0))],
        out_specs=[pl.BlockSpec((BN, Q), lambda i: (i, 0))] * 4 +
                  [pl.BlockSpec((BN, 1), lambda i: (i, 0))] * 2,
        out_shape=[jax.ShapeDtypeStruct((N, Q), jnp.float32)] * 4 +
                  [jax.ShapeDtypeStruct((N, 1), jnp.float32)] * 2,
    )(x, Wt, a1, a2)
    z0, z1, z2, z3, s2, d2 = zq

    s = jnp.pad(s2[:, 0], (0, NP - N))
    d = jnp.pad(d2[:, 0], (0, NP - N))
    src = jnp.pad(edge_index[0].reshape(NT, EPT),
                  ((0, 0), (0, EPTP - EPT))).reshape(NT, CH, G)
    dst = jnp.pad(edge_index[1].reshape(NT, EPT),
                  ((0, 0), (0, EPTP - EPT))).reshape(NT, CH, G)

    sc = pl.kernel(
        _sc_body,
        out_type=[jax.ShapeDtypeStruct((NP, Q), jnp.float32)] * 4,
        mesh=plsc.VectorSubcoreMesh(core_axis_name="c", subcore_axis_name="s"),
        compiler_params=pltpu.CompilerParams(needs_layout_passes=False,
                                             use_tc_tiling_on_sc=False),
        scratch_types=[
            pltpu.VMEM((NP,), jnp.float32),
            pltpu.VMEM((NP,), jnp.float32),
            pltpu.VMEM((CH, G), jnp.int32),
            pltpu.VMEM((CH, G), jnp.int32),
            pltpu.VMEM((CH, G), jnp.float32),
            pltpu.VMEM((RPT,), jnp.float32),
            pltpu.VMEM((G,), jnp.float32),
            pltpu.VMEM((G, Q), jnp.float32),
            pltpu.VMEM_SHARED((NP, Q), jnp.float32),
            pltpu.VMEM_SHARED((NP,), jnp.float32),
            pltpu.SemaphoreType.DMA,
        ],
    )
    o0, o1, o2, o3 = sc(z0, z1, z2, z3, s, d, src, dst)
    return jnp.concatenate([o0[:N], o1[:N], o2[:N], o3[:N]], axis=1)
